# baseline (speedup 1.0000x reference)
import functools

import jax
import jax.numpy as jnp
from jax.experimental import pallas as pl
from jax.experimental.pallas import tpu as pltpu

K = 20
A = 21
U = 64
L = 512
PDIM = L - K + 1
EPS = 1e-06
BB = 8


AP = 24


def _r_kernel(p_ref, q_ref, r_ref, rpad_ref):
    p = p_ref[...]
    m = jnp.max(p, axis=1, keepdims=True)
    e = jnp.exp(p - m)
    prob = e / jnp.sum(e, axis=1, keepdims=True)
    r = jnp.log(jnp.maximum(prob / q_ref[...], EPS))
    r_ref[...] = r
    rp = jnp.pad(r, ((0, 0), (0, AP - A), (0, 0)))
    rpad_ref[...] = rp.reshape(K * AP, U)


GRP = 5


def _conv_kernel(x_ref, rp_ref, z_ref, s_ref):
    for m in range(BB):
        xm = x_ref[m]
        z = jnp.zeros((L, U), dtype=jnp.float32)
        for g in range(K // GRP):
            xcg = jnp.concatenate(
                [xm[:, k:k + L] for k in range(g * GRP, (g + 1) * GRP)],
                axis=0)
            rg = rp_ref[pl.ds(g * GRP * AP, GRP * AP), :]
            z = z + jax.lax.dot_general(
                xcg, rg, (((0,), (0,)), ((), ())),
                preferred_element_type=jnp.float32)
        zv = z[:PDIM]
        z_ref[m] = zv
        s_ref[m, :] = jnp.max(zv, axis=0)


@functools.partial(jax.jit, static_argnums=())
def kernel(X, P_logit, Q):
    T, N, F, L_, A_ = X.shape
    B = T * N * F
    Xt = X.reshape(B, L_, A_).transpose(0, 2, 1)
    Xp = jnp.pad(Xt, ((0, 0), (0, AP - A), (0, K)))

    R, Rpad = pl.pallas_call(
        _r_kernel,
        out_shape=(
            jax.ShapeDtypeStruct((K, A, U), jnp.float32),
            jax.ShapeDtypeStruct((K * AP, U), jnp.float32),
        ),
    )(P_logit, Q.reshape(1, A, 1))

    Z, S = pl.pallas_call(
        _conv_kernel,
        grid=(B // BB,),
        in_specs=[
            pl.BlockSpec((BB, AP, L + K), lambda i: (i, 0, 0)),
            pl.BlockSpec((K * AP, U), lambda i: (0, 0)),
        ],
        out_specs=[
            pl.BlockSpec((BB, PDIM, U), lambda i: (i, 0, 0)),
            pl.BlockSpec((BB, U), lambda i: (i, 0)),
        ],
        out_shape=(
            jax.ShapeDtypeStruct((B, PDIM, U), jnp.float32),
            jax.ShapeDtypeStruct((B, U), jnp.float32),
        ),
        compiler_params=pltpu.CompilerParams(
            dimension_semantics=("arbitrary",)),
    )(Xp, Rpad)

    return (R, S.reshape(T, N, F, U), Z.reshape(T, N, F, PDIM, U))


# speedup vs baseline: 1.4759x; 1.4759x over previous
import functools

import jax
import jax.numpy as jnp
from jax.experimental import pallas as pl
from jax.experimental.pallas import tpu as pltpu

K = 20
A = 21
U = 64
L = 512
PDIM = L - K + 1
EPS = 1e-06
BB = 8


AP = 24


GRP = 5


def _r_kernel(p_ref, q_ref, r_ref, rt_ref):
    p = p_ref[...]
    m = jnp.max(p, axis=1, keepdims=True)
    e = jnp.exp(p - m)
    prob = e / jnp.sum(e, axis=1, keepdims=True)
    r = jnp.log(jnp.maximum(prob / q_ref[...], EPS))
    r_ref[...] = r
    rp = jnp.pad(r, ((0, 0), (0, AP - A), (0, 0))).reshape(K * AP, U)
    for g in range(K // GRP):
        rt_ref[g] = rp[g * GRP * AP:(g + 1) * GRP * AP, :].T


def _conv_kernel(x_ref, rt_ref, z_ref, s_ref):
    for m in range(BB):
        xm = x_ref[m]
        zt = jnp.zeros((U, L), dtype=jnp.float32)
        for g in range(K // GRP):
            xcg = jnp.concatenate(
                [xm[:, k:k + L] for k in range(g * GRP, (g + 1) * GRP)],
                axis=0)
            zt = zt + jax.lax.dot_general(
                rt_ref[g], xcg, (((1,), (0,)), ((), ())),
                preferred_element_type=jnp.float32)
        zv = zt.T[:PDIM]
        z_ref[m] = zv
        s_ref[m, :] = jnp.max(zv, axis=0)


@functools.partial(jax.jit, static_argnums=())
def kernel(X, P_logit, Q):
    T, N, F, L_, A_ = X.shape
    B = T * N * F
    Xt = X.reshape(B, L_, A_).transpose(0, 2, 1)
    Xp = jnp.pad(Xt, ((0, 0), (0, AP - A), (0, K)))

    R, Rt = pl.pallas_call(
        _r_kernel,
        out_shape=(
            jax.ShapeDtypeStruct((K, A, U), jnp.float32),
            jax.ShapeDtypeStruct((K // GRP, U, GRP * AP), jnp.float32),
        ),
    )(P_logit, Q.reshape(1, A, 1))

    Z, S = pl.pallas_call(
        _conv_kernel,
        grid=(B // BB,),
        in_specs=[
            pl.BlockSpec((BB, AP, L + K), lambda i: (i, 0, 0)),
            pl.BlockSpec((K // GRP, U, GRP * AP), lambda i: (0, 0, 0)),
        ],
        out_specs=[
            pl.BlockSpec((BB, PDIM, U), lambda i: (i, 0, 0)),
            pl.BlockSpec((BB, U), lambda i: (i, 0)),
        ],
        out_shape=(
            jax.ShapeDtypeStruct((B, PDIM, U), jnp.float32),
            jax.ShapeDtypeStruct((B, U), jnp.float32),
        ),
        compiler_params=pltpu.CompilerParams(
            dimension_semantics=("arbitrary",)),
    )(Xp, Rt)

    return (R, S.reshape(T, N, F, U), Z.reshape(T, N, F, PDIM, U))
